# Initial kernel scaffold; baseline (speedup 1.0000x reference)
import jax
import jax.numpy as jnp
from jax.experimental import pallas as pl


def kernel(q, k, v):
    raise NotImplementedError("write your pallas kernel here")


# speedup vs baseline: 2.0484x; 2.0484x over previous
import functools
import math

import jax
import jax.numpy as jnp
from jax.experimental import pallas as pl

_BLK = 512
_NEG = -1e9


def _pair_halves(x):
    n, d = x.shape
    y = x.reshape(n // 2, 2, d)
    return y[:, 0, :], y[:, 1, :]


def _pstep(Q, K, V, scale):
    c0Q, c1Q = _pair_halves(Q)
    c0K, c1K = _pair_halves(K)
    c0V, c1V = _pair_halves(V)
    pQ = 0.5 * (c0Q + c1Q)
    pK = 0.5 * (c0K + c1K)
    pV = 0.5 * (c0V + c1V)
    s_self = jnp.sum(pQ * pK, axis=-1, keepdims=True) * scale
    s0 = jnp.sum(pQ * c0K, axis=-1, keepdims=True) * scale
    s1 = jnp.sum(pQ * c1K, axis=-1, keepdims=True) * scale
    m = jnp.maximum(s_self, jnp.maximum(s0, s1))
    e_self = jnp.exp(s_self - m)
    e0 = jnp.exp(s0 - m)
    e1 = jnp.exp(s1 - m)
    den = e_self + e0 + e1 + 1e-9
    newK = (e_self * pK + e0 * c0K + e1 * c1K) / den
    newV = (e_self * pV + e0 * c0V + e1 * c1V) / den
    return pQ, newK, newV


def _pairswap(x):
    n, d = x.shape
    y = x.reshape(n // 2, 2, d)
    return jnp.concatenate([y[:, 1:2], y[:, 0:1]], axis=1).reshape(n, d)


def _sib_expand(x, out_rows):
    y = _pairswap(x)
    if y.shape[0] < out_rows:
        y = jnp.repeat(y, out_rows // y.shape[0], axis=0)
    return y


def _attn_one_head(q, k, v, TQ, TK, TV, n, lb, L, nblk, scale):
    blk, d = q.shape

    Ks, Vs = [], []
    Q, K, V = q, k, v
    for _ in range(1, lb):
        Q, K, V = _pstep(Q, K, V, scale)
        Ks.append(K)
        Vs.append(V)

    UK, UV = [TK], [TV]
    for _ in range(1, L - lb):
        TQ, TK, TV = _pstep(TQ, TK, TV, scale)
        UK.append(TK)
        UV.append(TV)

    row_id = jax.lax.broadcasted_iota(jnp.int32, (blk, 1), 0)

    scores = [jnp.sum(q * k, axis=-1, keepdims=True) * scale]
    vrows = [v]
    s0 = jnp.sum(q * _pairswap(k), axis=-1, keepdims=True) * scale
    scores.append(jnp.where((row_id & 1) == 0, _NEG, s0))
    vrows.append(_pairswap(v))
    for lvl in range(1, lb):
        Xk = _sib_expand(Ks[lvl - 1], blk)
        Xv = _sib_expand(Vs[lvl - 1], blk)
        s = jnp.sum(q * Xk, axis=-1, keepdims=True) * scale
        masked = ((row_id >> lvl) & 1) == 0
        scores.append(jnp.where(masked, _NEG, s))
        vrows.append(Xv)
    for u in range(L - lb):
        m_rows = nblk >> u
        sel = ((n >> u) ^ 1).astype(jnp.int32)
        rid = jax.lax.broadcasted_iota(jnp.int32, (m_rows, 1), 0)
        pick = (rid == sel).astype(jnp.float32)
        krow = jnp.sum(UK[u] * pick, axis=0, keepdims=True)
        vrow = jnp.sum(UV[u] * pick, axis=0, keepdims=True)
        s = jnp.sum(q * krow, axis=-1, keepdims=True) * scale
        blocked = ((n >> u) & 1) == 0
        s = jnp.where(blocked, jnp.full_like(s, _NEG), s)
        scores.append(s)
        vrows.append(jnp.broadcast_to(vrow, (blk, d)))

    m = scores[0]
    for s in scores[1:]:
        m = jnp.maximum(m, s)
    es = [jnp.exp(s - m) for s in scores]
    den = es[0]
    for e in es[1:]:
        den = den + e
    den = den + 1e-9
    acc = es[0] * vrows[0]
    for e, xv in zip(es[1:], vrows[1:]):
        acc = acc + e * xv
    return acc / den


def _build_top_kernel(q_ref, k_ref, v_ref, tq_ref, tk_ref, tv_ref, *,
                      levels, d, scale):
    n = pl.program_id(2)
    tqs, tks, tvs = [], [], []
    for h in range(2):
        sl = slice(h * d, (h + 1) * d)
        Q = q_ref[0][:, sl]
        K = k_ref[0][:, sl]
        V = v_ref[0][:, sl]
        for _ in range(levels):
            Q, K, V = _pstep(Q, K, V, scale)
        tqs.append(Q)
        tks.append(K)
        tvs.append(V)
    tq_ref[0, 0, pl.ds(n, 1)] = jnp.concatenate(tqs, axis=1)
    tk_ref[0, 0, pl.ds(n, 1)] = jnp.concatenate(tks, axis=1)
    tv_ref[0, 0, pl.ds(n, 1)] = jnp.concatenate(tvs, axis=1)


def _attn_kernel(q_ref, k_ref, v_ref, tq_ref, tk_ref, tv_ref, o_ref, *,
                 lb, L, nblk, d, scale):
    n = pl.program_id(2)
    outs = []
    for h in range(2):
        sl = slice(h * d, (h + 1) * d)
        outs.append(_attn_one_head(
            q_ref[0][:, sl], k_ref[0][:, sl], v_ref[0][:, sl],
            tq_ref[0, 0][:, sl], tk_ref[0, 0][:, sl], tv_ref[0, 0][:, sl],
            n, lb, L, nblk, scale))
    o_ref[0] = jnp.concatenate(outs, axis=1)


def kernel(q, k, v):
    B, S, H, D = q.shape
    L = int(math.log2(S))
    blk = min(_BLK, S)
    nblk = S // blk
    lb = int(math.log2(blk))
    scale = 1.0 / math.sqrt(D)
    H2 = H // 2

    qf = q.reshape(B, S, H * D)
    kf = k.reshape(B, S, H * D)
    vf = v.reshape(B, S, H * D)

    leaf_spec = pl.BlockSpec((1, blk, 2 * D), lambda b, h, n: (b, n, h))
    top_spec = pl.BlockSpec((1, 1, nblk, 2 * D), lambda b, h, n: (b, h, 0, 0))
    top_shape = jax.ShapeDtypeStruct((B, H2, nblk, 2 * D), jnp.float32)

    tq, tk, tv = pl.pallas_call(
        functools.partial(_build_top_kernel, levels=lb, d=D, scale=scale),
        grid=(B, H2, nblk),
        in_specs=[leaf_spec] * 3,
        out_specs=[top_spec] * 3,
        out_shape=[top_shape] * 3,
    )(qf, kf, vf)

    out = pl.pallas_call(
        functools.partial(_attn_kernel, lb=lb, L=L, nblk=nblk, d=D,
                          scale=scale),
        grid=(B, H2, nblk),
        in_specs=[leaf_spec] * 3 + [top_spec] * 3,
        out_specs=leaf_spec,
        out_shape=jax.ShapeDtypeStruct((B, S, H * D), jnp.float32),
    )(qf, kf, vf, tq, tk, tv)
    return out.reshape(B, S, H, D)


# speedup vs baseline: 5.4516x; 2.6614x over previous
import functools
import math

import jax
import jax.numpy as jnp
import numpy as np
from jax.experimental import pallas as pl

_BLK = 512
_NEG = -1e9


def _rsum(x, onesb):
    return jax.lax.dot_general(x, onesb, (((1,), (0,)), ((), ())),
                               preferred_element_type=jnp.float32)


def _spread(w, spr):
    return jax.lax.dot_general(w, spr, (((1,), (0,)), ((), ())),
                               preferred_element_type=jnp.float32)


def _pair_halves(x):
    n, d = x.shape
    y = x.reshape(n // 2, 2, d)
    return y[:, 0, :], y[:, 1, :]


def _pairswap(x):
    n, d = x.shape
    y = x.reshape(n // 2, 2, d)
    return jnp.concatenate([y[:, 1:2], y[:, 0:1]], axis=1).reshape(n, d)


def _pstep(c0Q, c1Q, c0K, c1K, c0V, c1V, onesb, spr, scale):
    pQ = 0.5 * (c0Q + c1Q)
    pK = 0.5 * (c0K + c1K)
    pV = 0.5 * (c0V + c1V)
    ss = _rsum(pQ * pK, onesb) * scale
    s0 = _rsum(pQ * c0K, onesb) * scale
    s1 = _rsum(pQ * c1K, onesb) * scale
    m = jnp.maximum(ss, jnp.maximum(s0, s1))
    es = jnp.exp(ss - m)
    e0 = jnp.exp(s0 - m)
    e1 = jnp.exp(s1 - m)
    den = es + e0 + e1 + 1e-9
    ws = _spread(es / den, spr)
    w0 = _spread(e0 / den, spr)
    w1 = _spread(e1 / den, spr)
    newK = ws * pK + w0 * c0K + w1 * c1K
    newV = ws * pV + w0 * c0V + w1 * c1V
    return pQ, newK, newV


def _pstep_packed(Q, K, V, onesb, spr, scale):
    (c0Q, c1Q) = _pair_halves(Q)
    (c0K, c1K) = _pair_halves(K)
    (c0V, c1V) = _pair_halves(V)
    return _pstep(c0Q, c1Q, c0K, c1K, c0V, c1V, onesb, spr, scale)


def _build_top_kernel(qe_ref, ke_ref, ve_ref, qo_ref, ko_ref, vo_ref,
                      onesb_ref, spr_ref, tq_ref, tk_ref, tv_ref, *,
                      levels, scale):
    n = pl.program_id(2)
    onesb = onesb_ref[...]
    spr = spr_ref[...]
    Q, K, V = _pstep(qe_ref[0], qo_ref[0], ke_ref[0], ko_ref[0],
                     ve_ref[0], vo_ref[0], onesb, spr, scale)
    for _ in range(2, levels + 1):
        Q, K, V = _pstep_packed(Q, K, V, onesb, spr, scale)
    tq_ref[0, 0, pl.ds(n, 1)] = Q
    tk_ref[0, 0, pl.ds(n, 1)] = K
    tv_ref[0, 0, pl.ds(n, 1)] = V


def _attn_kernel(qe_ref, ke_ref, ve_ref, qo_ref, ko_ref, vo_ref,
                 tq_ref, tk_ref, tv_ref, onesb_ref, spr_ref,
                 oe_ref, oo_ref, *, lb, L, nblk, scale):
    n = pl.program_id(2)
    onesb = onesb_ref[...]
    spr = spr_ref[...]

    Ks, Vs = [], []
    Q, K, V = _pstep(qe_ref[0], qo_ref[0], ke_ref[0], ko_ref[0],
                     ve_ref[0], vo_ref[0], onesb, spr, scale)
    Ks.append(K)
    Vs.append(V)
    for _ in range(2, lb):
        Q, K, V = _pstep_packed(Q, K, V, onesb, spr, scale)
        Ks.append(K)
        Vs.append(V)

    TQ = tq_ref[0, 0]
    TK = tk_ref[0, 0]
    TV = tv_ref[0, 0]
    UK, UV = [TK], [TV]
    for _ in range(1, L - lb):
        TQ, TK, TV = _pstep_packed(TQ, TK, TV, onesb, spr, scale)
        UK.append(TK)
        UV.append(TV)

    half = qe_ref[0].shape[0]
    row_id = jax.lax.broadcasted_iota(jnp.int32, (half, 1), 0)

    xk_list, xv_list, mask_list = [], [], []
    for lvl in range(1, lb):
        y_k = _pairswap(Ks[lvl - 1])
        y_v = _pairswap(Vs[lvl - 1])
        reps = half // y_k.shape[0]
        if reps > 1:
            y_k = jnp.repeat(y_k, reps, axis=0)
            y_v = jnp.repeat(y_v, reps, axis=0)
        xk_list.append(y_k)
        xv_list.append(y_v)
        mask_list.append(((row_id >> (lvl - 1)) & 1) == 0)
    up_k, up_v, up_blocked = [], [], []
    for u in range(L - lb):
        m_rows = nblk >> u
        sel = ((n >> u) ^ 1).astype(jnp.int32)
        rid = jax.lax.broadcasted_iota(jnp.int32, (m_rows, 1), 0)
        pick = (rid == sel).astype(jnp.float32)
        up_k.append(jnp.sum(UK[u] * pick, axis=0, keepdims=True))
        up_v.append(jnp.sum(UV[u] * pick, axis=0, keepdims=True))
        up_blocked.append(((n >> u) & 1) == 0)

    def stream(q, k_self, v_self, k_sib0, v_sib0):
        scores = [_rsum(q * k_self, onesb) * scale]
        vens = [v_self]
        if k_sib0 is not None:
            scores.append(_rsum(q * k_sib0, onesb) * scale)
            vens.append(v_sib0)
        for lvl in range(1, lb):
            s = _rsum(q * xk_list[lvl - 1], onesb) * scale
            scores.append(jnp.where(mask_list[lvl - 1], _NEG, s))
            vens.append(xv_list[lvl - 1])
        for u in range(L - lb):
            s = _rsum(q * up_k[u], onesb) * scale
            s = jnp.where(up_blocked[u], jnp.full_like(s, _NEG), s)
            scores.append(s)
            vens.append(up_v[u])
        m = scores[0]
        for s in scores[1:]:
            m = jnp.maximum(m, s)
        es = [jnp.exp(s - m) for s in scores]
        den = es[0]
        for e in es[1:]:
            den = den + e
        den = den + 1e-9
        acc = _spread(es[0] / den, spr) * vens[0]
        for e, xv in zip(es[1:], vens[1:]):
            acc = acc + _spread(e / den, spr) * xv
        return acc

    oe_ref[0] = stream(qe_ref[0], ke_ref[0], ve_ref[0], None, None)
    oo_ref[0] = stream(qo_ref[0], ko_ref[0], vo_ref[0], ke_ref[0], ve_ref[0])


def kernel(q, k, v):
    B, S, H, D = q.shape
    L = int(math.log2(S))
    blk = min(_BLK, S)
    nblk = S // blk
    lb = int(math.log2(blk))
    scale = 1.0 / math.sqrt(D)
    H2 = H // 2
    F = H * D
    S2 = S // 2
    half = blk // 2

    qf = q.reshape(B, S2, 2 * F)
    kf = k.reshape(B, S2, 2 * F)
    vf = v.reshape(B, S2, 2 * F)

    nb_f = F // 128
    espec = pl.BlockSpec((1, half, 128), lambda b, h, n: (b, n, h))
    ospec = pl.BlockSpec((1, half, 128),
                         lambda b, h, n, _o=nb_f: (b, n, _o + h))
    top_spec = pl.BlockSpec((1, 1, nblk, 128), lambda b, h, n: (b, h, 0, 0))
    top_shape = jax.ShapeDtypeStruct((B, H2, nblk, 128), jnp.float32)
    onesb = jnp.asarray(np.kron(np.eye(2), np.ones((64, 1))),
                        dtype=jnp.float32)
    spr = jnp.asarray(np.kron(np.eye(2), np.ones((1, 64))),
                      dtype=jnp.float32)
    onesb_spec = pl.BlockSpec((128, 2), lambda b, h, n: (0, 0))
    spr_spec = pl.BlockSpec((2, 128), lambda b, h, n: (0, 0))

    tq, tk, tv = pl.pallas_call(
        functools.partial(_build_top_kernel, levels=lb, scale=scale),
        grid=(B, H2, nblk),
        in_specs=[espec, espec, espec, ospec, ospec, ospec,
                  onesb_spec, spr_spec],
        out_specs=[top_spec] * 3,
        out_shape=[top_shape] * 3,
    )(qf, kf, vf, qf, kf, vf, onesb, spr)

    oe, oo = pl.pallas_call(
        functools.partial(_attn_kernel, lb=lb, L=L, nblk=nblk, scale=scale),
        grid=(B, H2, nblk),
        in_specs=[espec, espec, espec, ospec, ospec, ospec,
                  top_spec, top_spec, top_spec, onesb_spec, spr_spec],
        out_specs=[espec, espec],
        out_shape=[jax.ShapeDtypeStruct((B, S2, F), jnp.float32)] * 2,
    )(qf, kf, vf, qf, kf, vf, tq, tk, tv, onesb, spr)

    out = jnp.stack([oe, oo], axis=2).reshape(B, S, H, D)
    return out
